# baseline (device time: 79554 ns/iter reference)
import jax
import jax.numpy as jnp
from jax import lax
from jax.experimental import pallas as pl
from jax.experimental.pallas import tpu as pltpu

N_DEV = 4
SQ = 1024
SKV = 1024
HQ_LOCAL = 8
DH = 128
D_MODEL = 1024
CHUNK = SQ // N_DEV
SCALE = 0.08838834764831843
MESH = pl.DeviceIdType.MESH


def _mod4(v):
    return lax.rem(v + 2 * N_DEV, N_DEV)


def _body(*refs):
    (x_ref, wq_ref), k_refs, v_refs, (wo_ref, out_ref) = (
        refs[0:2], refs[2:2 + HQ_LOCAL], refs[10:10 + HQ_LOCAL], refs[18:20])
    (sendbuf_ref, srecv_ref, mychunk_ref, brecv_ref, pown_ref,
     ssend_sems, srecv_sems, bsend_sems, brecv_sems) = refs[20:]
    my = lax.axis_index("i")

    barrier_sem = pltpu.get_barrier_semaphore()
    for j in range(1, N_DEV):
        pl.semaphore_signal(
            barrier_sem, inc=1,
            device_id=(_mod4(my + j),), device_id_type=MESH,
        )
    pl.semaphore_wait(barrier_sem, N_DEV - 1)

    ki = lax.broadcasted_iota(jnp.int32, (CHUNK, SKV), 1)
    qi_rel = lax.broadcasted_iota(jnp.int32, (CHUNK, SKV), 0)
    DOT11 = (((1,), (1,)), ((), ()))

    def partial_chunk(c):
        rows = pl.ds(c * CHUNK, CHUNK)
        q = (jnp.dot(x_ref[rows, :], wq_ref[:, :],
                     preferred_element_type=jnp.float32)
             * SCALE).astype(jnp.bfloat16)
        qi = qi_rel + c * CHUNK
        mask = (jnp.abs(qi - ki) <= 128) | (ki < 32) | (qi < 32)
        ctx = []
        for h in range(HQ_LOCAL):
            qh = q[:, h * DH:(h + 1) * DH]
            s = lax.dot_general(
                qh, k_refs[h][:, :],
                dimension_numbers=DOT11,
                preferred_element_type=jnp.float32,
            )
            w = jnp.exp(jnp.where(mask, s, -1e9))
            w = w * (1.0 / jnp.sum(w, axis=-1, keepdims=True))
            ch = jnp.dot(w.astype(jnp.bfloat16), v_refs[h][:, :],
                         preferred_element_type=jnp.float32)
            ctx.append(ch.astype(jnp.bfloat16))
        ctx = jnp.concatenate(ctx, axis=1)
        return jnp.dot(ctx, wo_ref[:, :], preferred_element_type=jnp.float32)

    scatter = []
    for j in range(N_DEV - 1):
        tgt = _mod4(my + 1 + j)
        sendbuf_ref[j, :, :] = partial_chunk(tgt).astype(jnp.bfloat16)
        rdma = pltpu.make_async_remote_copy(
            src_ref=sendbuf_ref.at[j],
            dst_ref=srecv_ref.at[2 - j],
            send_sem=ssend_sems.at[j],
            recv_sem=srecv_sems.at[2 - j],
            device_id=(tgt,), device_id_type=MESH,
        )
        rdma.start()
        scatter.append(rdma)

    pown_ref[:, :] = partial_chunk(my)

    acc = pown_ref[:, :]
    for i in range(N_DEV - 1):
        recv = pltpu.make_async_remote_copy(
            src_ref=sendbuf_ref.at[0],
            dst_ref=srecv_ref.at[i],
            send_sem=ssend_sems.at[0],
            recv_sem=srecv_sems.at[i],
            device_id=(my,), device_id_type=MESH,
        )
        recv.wait_recv()
        acc = acc + srecv_ref[i].astype(jnp.float32)
    mychunk_ref[:, :] = acc.astype(jnp.bfloat16)
    out_ref[pl.ds(my * CHUNK, CHUNK), :] = mychunk_ref[:, :]

    bcasts = []
    for j in range(N_DEV - 1):
        tgt = _mod4(my + 1 + j)
        rdma = pltpu.make_async_remote_copy(
            src_ref=mychunk_ref,
            dst_ref=brecv_ref.at[2 - j],
            send_sem=bsend_sems.at[j],
            recv_sem=brecv_sems.at[2 - j],
            device_id=(tgt,), device_id_type=MESH,
        )
        rdma.start()
        bcasts.append(rdma)

    for i in range(N_DEV - 1):
        recv = pltpu.make_async_remote_copy(
            src_ref=mychunk_ref,
            dst_ref=brecv_ref.at[i],
            send_sem=bsend_sems.at[0],
            recv_sem=brecv_sems.at[i],
            device_id=(my,), device_id_type=MESH,
        )
        recv.wait_recv()
        src_chip = _mod4(my + 1 + i)
        out_ref[pl.ds(src_chip * CHUNK, CHUNK), :] = brecv_ref[i, :, :]

    for rdma in scatter + bcasts:
        rdma.wait_send()


def kernel(x, Wq, K_ext, V_ext, Wo):
    my = lax.axis_index("i")

    xb = x[0].astype(jnp.bfloat16)
    Wqb = Wq.astype(jnp.bfloat16)
    Wob = Wo.astype(jnp.bfloat16)
    Kh = [lax.dynamic_slice_in_dim(K_ext[0], my * HQ_LOCAL + h, 1, axis=1)
          [:, 0, :].astype(jnp.bfloat16) for h in range(HQ_LOCAL)]
    Vh = [lax.dynamic_slice_in_dim(V_ext[0], my * HQ_LOCAL + h, 1, axis=1)
          [:, 0, :].astype(jnp.bfloat16) for h in range(HQ_LOCAL)]

    out = pl.pallas_call(
        _body,
        out_shape=jax.ShapeDtypeStruct((SQ, D_MODEL), jnp.bfloat16),
        in_specs=[pl.BlockSpec(memory_space=pltpu.VMEM)] * (3 + 2 * HQ_LOCAL),
        out_specs=pl.BlockSpec(memory_space=pltpu.VMEM),
        scratch_shapes=[
            pltpu.VMEM((N_DEV - 1, CHUNK, D_MODEL), jnp.bfloat16),
            pltpu.VMEM((N_DEV - 1, CHUNK, D_MODEL), jnp.bfloat16),
            pltpu.VMEM((CHUNK, D_MODEL), jnp.bfloat16),
            pltpu.VMEM((N_DEV - 1, CHUNK, D_MODEL), jnp.bfloat16),
            pltpu.VMEM((CHUNK, D_MODEL), jnp.float32),
            pltpu.SemaphoreType.DMA((N_DEV - 1,)),
            pltpu.SemaphoreType.DMA((N_DEV - 1,)),
            pltpu.SemaphoreType.DMA((N_DEV - 1,)),
            pltpu.SemaphoreType.DMA((N_DEV - 1,)),
        ],
        compiler_params=pltpu.CompilerParams(collective_id=0),
    )(xb, Wqb, *Kh, *Vh, Wob)
    return out[None]


# device time: 55444 ns/iter; 1.4349x vs baseline; 1.4349x over previous
import jax
import jax.numpy as jnp
from jax import lax
from jax.experimental import pallas as pl
from jax.experimental.pallas import tpu as pltpu

N_DEV = 4
SQ = 1024
SKV = 1024
HQ_LOCAL = 8
DH = 128
D_MODEL = 1024
CHUNK = SQ // N_DEV
SCALE = 0.08838834764831843
MESH = pl.DeviceIdType.MESH


def _mod4(v):
    return lax.rem(v + 2 * N_DEV, N_DEV)


def _body(x_ref, wq_ref, k_ref, v_ref, wo_ref, out_ref,
          sendbuf_ref, srecv_ref, mychunk_ref, brecv_ref, pown_ref,
          ssend_sems, srecv_sems, bsend_sems, brecv_sems):
    my = lax.axis_index("i")

    barrier_sem = pltpu.get_barrier_semaphore()
    for j in range(1, N_DEV):
        pl.semaphore_signal(
            barrier_sem, inc=1,
            device_id=(_mod4(my + j),), device_id_type=MESH,
        )
    pl.semaphore_wait(barrier_sem, N_DEV - 1)

    ki = lax.broadcasted_iota(jnp.int32, (CHUNK, SKV), 1)
    qi_rel = lax.broadcasted_iota(jnp.int32, (CHUNK, SKV), 0)
    DOT11 = (((1,), (1,)), ((), ()))

    def partial_chunk(c):
        rows = pl.ds(c * CHUNK, CHUNK)
        q = (jnp.dot(x_ref[rows, :], wq_ref[:, :],
                     preferred_element_type=jnp.float32)
             * SCALE).astype(jnp.bfloat16)
        qi = qi_rel + c * CHUNK
        mask = (jnp.abs(qi - ki) <= 128) | (ki < 32) | (qi < 32)
        ctx = []
        for h in range(HQ_LOCAL):
            qh = q[:, h * DH:(h + 1) * DH]
            s = lax.dot_general(
                qh, k_ref[h],
                dimension_numbers=DOT11,
                preferred_element_type=jnp.float32,
            )
            w = jnp.exp(jnp.where(mask, s, -1e9))
            w = w * (1.0 / jnp.sum(w, axis=-1, keepdims=True))
            ch = jnp.dot(w.astype(jnp.bfloat16), v_ref[h],
                         preferred_element_type=jnp.float32)
            ctx.append(ch.astype(jnp.bfloat16))
        ctx = jnp.concatenate(ctx, axis=1)
        return jnp.dot(ctx, wo_ref[:, :], preferred_element_type=jnp.float32)

    scatter = []
    for j in range(N_DEV - 1):
        tgt = _mod4(my + 1 + j)
        sendbuf_ref[j, :, :] = partial_chunk(tgt).astype(jnp.bfloat16)
        rdma = pltpu.make_async_remote_copy(
            src_ref=sendbuf_ref.at[j],
            dst_ref=srecv_ref.at[2 - j],
            send_sem=ssend_sems.at[j],
            recv_sem=srecv_sems.at[2 - j],
            device_id=(tgt,), device_id_type=MESH,
        )
        rdma.start()
        scatter.append(rdma)

    pown_ref[:, :] = partial_chunk(my)

    acc = pown_ref[:, :]
    for i in range(N_DEV - 1):
        recv = pltpu.make_async_remote_copy(
            src_ref=sendbuf_ref.at[0],
            dst_ref=srecv_ref.at[i],
            send_sem=ssend_sems.at[0],
            recv_sem=srecv_sems.at[i],
            device_id=(my,), device_id_type=MESH,
        )
        recv.wait_recv()
        acc = acc + srecv_ref[i].astype(jnp.float32)
    mychunk_ref[:, :] = acc.astype(jnp.bfloat16)
    out_ref[pl.ds(my * CHUNK, CHUNK), :] = mychunk_ref[:, :]

    bcasts = []
    for j in range(N_DEV - 1):
        tgt = _mod4(my + 1 + j)
        rdma = pltpu.make_async_remote_copy(
            src_ref=mychunk_ref,
            dst_ref=brecv_ref.at[2 - j],
            send_sem=bsend_sems.at[j],
            recv_sem=brecv_sems.at[2 - j],
            device_id=(tgt,), device_id_type=MESH,
        )
        rdma.start()
        bcasts.append(rdma)

    for i in range(N_DEV - 1):
        recv = pltpu.make_async_remote_copy(
            src_ref=mychunk_ref,
            dst_ref=brecv_ref.at[i],
            send_sem=bsend_sems.at[0],
            recv_sem=brecv_sems.at[i],
            device_id=(my,), device_id_type=MESH,
        )
        recv.wait_recv()
        src_chip = _mod4(my + 1 + i)
        out_ref[pl.ds(src_chip * CHUNK, CHUNK), :] = brecv_ref[i, :, :]

    for rdma in scatter + bcasts:
        rdma.wait_send()


def kernel(x, Wq, K_ext, V_ext, Wo):
    my = lax.axis_index("i")

    xb = x[0].astype(jnp.bfloat16)
    Wqb = Wq.astype(jnp.bfloat16)
    Wob = Wo.astype(jnp.bfloat16)
    Kh = lax.dynamic_slice_in_dim(K_ext[0], my * HQ_LOCAL, HQ_LOCAL, axis=1)
    Vh = lax.dynamic_slice_in_dim(V_ext[0], my * HQ_LOCAL, HQ_LOCAL, axis=1)
    Kh = jnp.transpose(Kh, (1, 0, 2)).astype(jnp.bfloat16)
    Vh = jnp.transpose(Vh, (1, 0, 2)).astype(jnp.bfloat16)

    out = pl.pallas_call(
        _body,
        out_shape=jax.ShapeDtypeStruct((SQ, D_MODEL), jnp.bfloat16),
        in_specs=[pl.BlockSpec(memory_space=pltpu.VMEM)] * 5,
        out_specs=pl.BlockSpec(memory_space=pltpu.VMEM),
        scratch_shapes=[
            pltpu.VMEM((N_DEV - 1, CHUNK, D_MODEL), jnp.bfloat16),
            pltpu.VMEM((N_DEV - 1, CHUNK, D_MODEL), jnp.bfloat16),
            pltpu.VMEM((CHUNK, D_MODEL), jnp.bfloat16),
            pltpu.VMEM((N_DEV - 1, CHUNK, D_MODEL), jnp.bfloat16),
            pltpu.VMEM((CHUNK, D_MODEL), jnp.float32),
            pltpu.SemaphoreType.DMA((N_DEV - 1,)),
            pltpu.SemaphoreType.DMA((N_DEV - 1,)),
            pltpu.SemaphoreType.DMA((N_DEV - 1,)),
            pltpu.SemaphoreType.DMA((N_DEV - 1,)),
        ],
        compiler_params=pltpu.CompilerParams(collective_id=0),
    )(xb, Wqb, Kh, Vh, Wob)
    return out[None]


# device time: 51956 ns/iter; 1.5312x vs baseline; 1.0671x over previous
import jax
import jax.numpy as jnp
from jax import lax
from jax.experimental import pallas as pl
from jax.experimental.pallas import tpu as pltpu

N_DEV = 4
SQ = 1024
SKV = 1024
HQ_LOCAL = 8
DH = 128
D_MODEL = 1024
CHUNK = SQ // N_DEV
SCALE = 0.08838834764831843
MESH = pl.DeviceIdType.MESH


def _mod4(v):
    return lax.rem(v + 2 * N_DEV, N_DEV)


def _body(x_ref, wq_ref, k_hbm, v_hbm, wo_ref, out_ref,
          ktmp_ref, vtmp_ref, k_ref, v_ref,
          sendbuf_ref, srecv_ref, mychunk_ref, brecv_ref, pown_ref,
          kv_sems, ssend_sems, srecv_sems, bsend_sems, brecv_sems):
    my = lax.axis_index("i")

    kv_copies = []
    for h in range(HQ_LOCAL):
        kc = pltpu.make_async_copy(
            k_hbm.at[:, my * HQ_LOCAL + h, :], ktmp_ref.at[h],
            kv_sems.at[h])
        vc = pltpu.make_async_copy(
            v_hbm.at[:, my * HQ_LOCAL + h, :], vtmp_ref.at[h],
            kv_sems.at[HQ_LOCAL + h])
        kc.start()
        vc.start()
        kv_copies += [kc, vc]

    barrier_sem = pltpu.get_barrier_semaphore()
    for j in range(1, N_DEV):
        pl.semaphore_signal(
            barrier_sem, inc=1,
            device_id=(_mod4(my + j),), device_id_type=MESH,
        )
    pl.semaphore_wait(barrier_sem, N_DEV - 1)

    for cpy in kv_copies:
        cpy.wait()
    for h in range(HQ_LOCAL):
        k_ref[h, :, :] = ktmp_ref[h].astype(jnp.bfloat16)
        v_ref[h, :, :] = vtmp_ref[h].astype(jnp.bfloat16)

    ki = lax.broadcasted_iota(jnp.int32, (CHUNK, SKV), 1)
    qi_rel = lax.broadcasted_iota(jnp.int32, (CHUNK, SKV), 0)
    DOT11 = (((1,), (1,)), ((), ()))

    def partial_chunk(c):
        rows = pl.ds(c * CHUNK, CHUNK)
        q = (jnp.dot(x_ref[rows, :], wq_ref[:, :],
                     preferred_element_type=jnp.float32)
             * SCALE).astype(jnp.bfloat16)
        qi = qi_rel + c * CHUNK
        mask = (jnp.abs(qi - ki) <= 128) | (ki < 32) | (qi < 32)
        ctx = []
        for h in range(HQ_LOCAL):
            qh = q[:, h * DH:(h + 1) * DH]
            s = lax.dot_general(
                qh, k_ref[h],
                dimension_numbers=DOT11,
                preferred_element_type=jnp.float32,
            )
            w = jnp.exp(jnp.where(mask, s, -1e9))
            w = w * (1.0 / jnp.sum(w, axis=-1, keepdims=True))
            ch = jnp.dot(w.astype(jnp.bfloat16), v_ref[h],
                         preferred_element_type=jnp.float32)
            ctx.append(ch.astype(jnp.bfloat16))
        ctx = jnp.concatenate(ctx, axis=1)
        return jnp.dot(ctx, wo_ref[:, :], preferred_element_type=jnp.float32)

    scatter = []
    for j in range(N_DEV - 1):
        tgt = _mod4(my + 1 + j)
        sendbuf_ref[j, :, :] = partial_chunk(tgt).astype(jnp.bfloat16)
        rdma = pltpu.make_async_remote_copy(
            src_ref=sendbuf_ref.at[j],
            dst_ref=srecv_ref.at[2 - j],
            send_sem=ssend_sems.at[j],
            recv_sem=srecv_sems.at[2 - j],
            device_id=(tgt,), device_id_type=MESH,
        )
        rdma.start()
        scatter.append(rdma)

    pown_ref[:, :] = partial_chunk(my)

    acc = pown_ref[:, :]
    for i in range(N_DEV - 1):
        recv = pltpu.make_async_remote_copy(
            src_ref=sendbuf_ref.at[0],
            dst_ref=srecv_ref.at[i],
            send_sem=ssend_sems.at[0],
            recv_sem=srecv_sems.at[i],
            device_id=(my,), device_id_type=MESH,
        )
        recv.wait_recv()
        acc = acc + srecv_ref[i].astype(jnp.float32)
    mychunk_ref[:, :] = acc.astype(jnp.bfloat16)
    out_ref[pl.ds(my * CHUNK, CHUNK), :] = mychunk_ref[:, :]

    bcasts = []
    for j in range(N_DEV - 1):
        tgt = _mod4(my + 1 + j)
        rdma = pltpu.make_async_remote_copy(
            src_ref=mychunk_ref,
            dst_ref=brecv_ref.at[2 - j],
            send_sem=bsend_sems.at[j],
            recv_sem=brecv_sems.at[2 - j],
            device_id=(tgt,), device_id_type=MESH,
        )
        rdma.start()
        bcasts.append(rdma)

    for i in range(N_DEV - 1):
        recv = pltpu.make_async_remote_copy(
            src_ref=mychunk_ref,
            dst_ref=brecv_ref.at[i],
            send_sem=bsend_sems.at[0],
            recv_sem=brecv_sems.at[i],
            device_id=(my,), device_id_type=MESH,
        )
        recv.wait_recv()
        src_chip = _mod4(my + 1 + i)
        out_ref[pl.ds(src_chip * CHUNK, CHUNK), :] = brecv_ref[i, :, :]

    for rdma in scatter + bcasts:
        rdma.wait_send()


def kernel(x, Wq, K_ext, V_ext, Wo):
    my = lax.axis_index("i")

    xb = x[0].astype(jnp.bfloat16)
    Wqb = Wq.astype(jnp.bfloat16)
    Wob = Wo.astype(jnp.bfloat16)

    out = pl.pallas_call(
        _body,
        out_shape=jax.ShapeDtypeStruct((SQ, D_MODEL), jnp.bfloat16),
        in_specs=[
            pl.BlockSpec(memory_space=pltpu.VMEM),
            pl.BlockSpec(memory_space=pltpu.VMEM),
            pl.BlockSpec(memory_space=pl.ANY),
            pl.BlockSpec(memory_space=pl.ANY),
            pl.BlockSpec(memory_space=pltpu.VMEM),
        ],
        out_specs=pl.BlockSpec(memory_space=pltpu.VMEM),
        scratch_shapes=[
            pltpu.VMEM((HQ_LOCAL, SKV, DH), jnp.float32),
            pltpu.VMEM((HQ_LOCAL, SKV, DH), jnp.float32),
            pltpu.VMEM((HQ_LOCAL, SKV, DH), jnp.bfloat16),
            pltpu.VMEM((HQ_LOCAL, SKV, DH), jnp.bfloat16),
            pltpu.VMEM((N_DEV - 1, CHUNK, D_MODEL), jnp.bfloat16),
            pltpu.VMEM((N_DEV - 1, CHUNK, D_MODEL), jnp.bfloat16),
            pltpu.VMEM((CHUNK, D_MODEL), jnp.bfloat16),
            pltpu.VMEM((N_DEV - 1, CHUNK, D_MODEL), jnp.bfloat16),
            pltpu.VMEM((CHUNK, D_MODEL), jnp.float32),
            pltpu.SemaphoreType.DMA((2 * HQ_LOCAL,)),
            pltpu.SemaphoreType.DMA((N_DEV - 1,)),
            pltpu.SemaphoreType.DMA((N_DEV - 1,)),
            pltpu.SemaphoreType.DMA((N_DEV - 1,)),
            pltpu.SemaphoreType.DMA((N_DEV - 1,)),
        ],
        compiler_params=pltpu.CompilerParams(collective_id=0),
    )(xb, Wqb, K_ext[0], V_ext[0], Wob)
    return out[None]


# device time: 51950 ns/iter; 1.5314x vs baseline; 1.0001x over previous
import jax
import jax.numpy as jnp
from jax import lax
from jax.experimental import pallas as pl
from jax.experimental.pallas import tpu as pltpu

N_DEV = 4
SQ = 1024
SKV = 1024
HQ_LOCAL = 8
DH = 128
D_MODEL = 1024
CHUNK = SQ // N_DEV
SCALE = 0.08838834764831843
MESH = pl.DeviceIdType.MESH


def _mod4(v):
    return lax.rem(v + 2 * N_DEV, N_DEV)


def _body(x_ref, wq_ref, k_hbm, v_hbm, wo_ref, out_ref,
          ktmp_ref, vtmp_ref, k_ref, v_ref,
          sendbuf_ref, srecv_ref, mychunk_ref, brecv_ref, pown_ref,
          kv_sems, ssend_sems, srecv_sems, bsend_sems, brecv_sems):
    my = lax.axis_index("i")

    kv_copies = []
    for h in range(HQ_LOCAL):
        kc = pltpu.make_async_copy(
            k_hbm.at[:, my * HQ_LOCAL + h, :], ktmp_ref.at[h],
            kv_sems.at[h])
        vc = pltpu.make_async_copy(
            v_hbm.at[:, my * HQ_LOCAL + h, :], vtmp_ref.at[h],
            kv_sems.at[HQ_LOCAL + h])
        kc.start()
        vc.start()
        kv_copies += [kc, vc]

    barrier_sem = pltpu.get_barrier_semaphore()
    for j in range(1, N_DEV):
        pl.semaphore_signal(
            barrier_sem, inc=1,
            device_id=(_mod4(my + j),), device_id_type=MESH,
        )
    pl.semaphore_wait(barrier_sem, N_DEV - 1)

    for cpy in kv_copies:
        cpy.wait()
    for h in range(HQ_LOCAL):
        k_ref[h, :, :] = ktmp_ref[h].astype(jnp.bfloat16)
        v_ref[h, :, :] = vtmp_ref[h].astype(jnp.bfloat16)

    ki = lax.broadcasted_iota(jnp.int32, (CHUNK, SKV), 1)
    qi_rel = lax.broadcasted_iota(jnp.int32, (CHUNK, SKV), 0)
    DOT11 = (((1,), (1,)), ((), ()))

    def partial_chunk(c):
        rows = pl.ds(c * CHUNK, CHUNK)
        q = (jnp.dot(x_ref[rows, :], wq_ref[:, :],
                     preferred_element_type=jnp.float32)
             * SCALE).astype(jnp.bfloat16)
        qi = qi_rel + c * CHUNK
        mask = (jnp.abs(qi - ki) <= 128) | (ki < 32) | (qi < 32)
        ctx = []
        for h in range(HQ_LOCAL):
            qh = q[:, h * DH:(h + 1) * DH]
            s = lax.dot_general(
                qh, k_ref[h],
                dimension_numbers=DOT11,
                preferred_element_type=jnp.float32,
            )
            w = jnp.exp(jnp.where(mask, s, -1e9))
            w = w * (1.0 / jnp.sum(w, axis=-1, keepdims=True))
            ch = jnp.dot(w.astype(jnp.bfloat16), v_ref[h],
                         preferred_element_type=jnp.float32)
            ctx.append(ch.astype(jnp.bfloat16))
        ctx = jnp.concatenate(ctx, axis=1)
        return jnp.dot(ctx, wo_ref[:, :], preferred_element_type=jnp.float32)

    scatter = []
    for j in range(N_DEV - 1):
        tgt = _mod4(my + 1 + j)
        sendbuf_ref[j, :, :] = partial_chunk(tgt).astype(jnp.bfloat16)
        rdma = pltpu.make_async_remote_copy(
            src_ref=sendbuf_ref.at[j],
            dst_ref=srecv_ref.at[2 - j],
            send_sem=ssend_sems.at[j],
            recv_sem=srecv_sems.at[2 - j],
            device_id=(tgt,), device_id_type=MESH,
        )
        rdma.start()
        scatter.append(rdma)

    pown_ref[:, :] = partial_chunk(my)

    acc = pown_ref[:, :]
    for i in range(N_DEV - 1):
        recv = pltpu.make_async_remote_copy(
            src_ref=sendbuf_ref.at[0],
            dst_ref=srecv_ref.at[i],
            send_sem=ssend_sems.at[0],
            recv_sem=srecv_sems.at[i],
            device_id=(my,), device_id_type=MESH,
        )
        recv.wait_recv()
        acc = acc + srecv_ref[i].astype(jnp.float32)
    mychunk_ref[:, :] = acc.astype(jnp.bfloat16)
    out_ref[0, pl.ds(my * CHUNK, CHUNK), :] = mychunk_ref[:, :]

    bcasts = []
    for j in range(N_DEV - 1):
        tgt = _mod4(my + 1 + j)
        rdma = pltpu.make_async_remote_copy(
            src_ref=mychunk_ref,
            dst_ref=brecv_ref.at[2 - j],
            send_sem=bsend_sems.at[j],
            recv_sem=brecv_sems.at[2 - j],
            device_id=(tgt,), device_id_type=MESH,
        )
        rdma.start()
        bcasts.append(rdma)

    for i in range(N_DEV - 1):
        recv = pltpu.make_async_remote_copy(
            src_ref=mychunk_ref,
            dst_ref=brecv_ref.at[i],
            send_sem=bsend_sems.at[0],
            recv_sem=brecv_sems.at[i],
            device_id=(my,), device_id_type=MESH,
        )
        recv.wait_recv()
        src_chip = _mod4(my + 1 + i)
        out_ref[0, pl.ds(src_chip * CHUNK, CHUNK), :] = brecv_ref[i, :, :]

    for rdma in scatter + bcasts:
        rdma.wait_send()


def kernel(x, Wq, K_ext, V_ext, Wo):
    my = lax.axis_index("i")

    xb = x[0].astype(jnp.bfloat16)
    Wqb = Wq.astype(jnp.bfloat16)
    Wob = Wo.astype(jnp.bfloat16)

    out = pl.pallas_call(
        _body,
        out_shape=jax.ShapeDtypeStruct((1, SQ, D_MODEL), jnp.bfloat16),
        in_specs=[
            pl.BlockSpec(memory_space=pltpu.VMEM),
            pl.BlockSpec(memory_space=pltpu.VMEM),
            pl.BlockSpec(memory_space=pl.ANY),
            pl.BlockSpec(memory_space=pl.ANY),
            pl.BlockSpec(memory_space=pltpu.VMEM),
        ],
        out_specs=pl.BlockSpec(memory_space=pltpu.VMEM),
        scratch_shapes=[
            pltpu.VMEM((HQ_LOCAL, SKV, DH), jnp.float32),
            pltpu.VMEM((HQ_LOCAL, SKV, DH), jnp.float32),
            pltpu.VMEM((HQ_LOCAL, SKV, DH), jnp.bfloat16),
            pltpu.VMEM((HQ_LOCAL, SKV, DH), jnp.bfloat16),
            pltpu.VMEM((N_DEV - 1, CHUNK, D_MODEL), jnp.bfloat16),
            pltpu.VMEM((N_DEV - 1, CHUNK, D_MODEL), jnp.bfloat16),
            pltpu.VMEM((CHUNK, D_MODEL), jnp.bfloat16),
            pltpu.VMEM((N_DEV - 1, CHUNK, D_MODEL), jnp.bfloat16),
            pltpu.VMEM((CHUNK, D_MODEL), jnp.float32),
            pltpu.SemaphoreType.DMA((2 * HQ_LOCAL,)),
            pltpu.SemaphoreType.DMA((N_DEV - 1,)),
            pltpu.SemaphoreType.DMA((N_DEV - 1,)),
            pltpu.SemaphoreType.DMA((N_DEV - 1,)),
            pltpu.SemaphoreType.DMA((N_DEV - 1,)),
        ],
        compiler_params=pltpu.CompilerParams(collective_id=0),
    )(xb, Wqb, K_ext[0], V_ext[0], Wob)
    return out
